# baseline (speedup 1.0000x reference)
import functools

import jax
import jax.numpy as jnp
from jax import lax
from jax.experimental import pallas as pl
from jax.experimental.pallas import tpu as pltpu
from jax.experimental.pallas import tpu_sc as plsc

VOCAB = 100000
EMBED_DIM = 64
BATCH = 1024
CTX_WIN = 20

NC = 2
NS = 16
NW = NC * NS
B_PER_W = BATCH // NW
ROWS_PER_W = B_PER_W * CTX_WIN
IDX_CHUNK = 128
N_CHUNKS = ROWS_PER_W // IDX_CHUNK

VT = 512
NVT = (VOCAB + VT - 1) // VT

_NEG_INF = float("-inf")


def _sc_gather_sum_body(idx_hbm, table_hbm, out_hbm, idx_v, rows_v, h_v, sem):
    wid = lax.axis_index("s") * NC + lax.axis_index("c")
    pltpu.sync_copy(idx_hbm.at[wid], idx_v)
    copies = [
        pltpu.async_copy(
            table_hbm.at[idx_v.at[j]],
            rows_v.at[pl.ds(j * IDX_CHUNK, IDX_CHUNK)],
            sem,
        )
        for j in range(N_CHUNKS)
    ]
    for c in copies:
        c.wait()

    def row_body(r, carry):
        base = r * CTX_WIN
        for dch in range(EMBED_DIM // 16):
            acc = jnp.zeros((16,), jnp.float32)
            for c in range(CTX_WIN):
                acc = acc + rows_v[base + c, pl.ds(dch * 16, 16)]
            h_v[r, pl.ds(dch * 16, 16)] = acc
        return carry

    lax.fori_loop(0, B_PER_W, row_body, 0)
    pltpu.sync_copy(h_v, out_hbm.at[pl.ds(wid * B_PER_W, B_PER_W)])


@functools.cache
def _sc_gather_sum():
    return pl.kernel(
        _sc_gather_sum_body,
        mesh=plsc.VectorSubcoreMesh(core_axis_name="c", subcore_axis_name="s"),
        out_type=jax.ShapeDtypeStruct((BATCH, EMBED_DIM), jnp.float32),
        scratch_types=[
            pltpu.VMEM((N_CHUNKS, IDX_CHUNK), jnp.int32),
            pltpu.VMEM((ROWS_PER_W, EMBED_DIM), jnp.float32),
            pltpu.VMEM((B_PER_W, EMBED_DIM), jnp.float32),
            pltpu.SemaphoreType.DMA,
        ],
        compiler_params=pltpu.CompilerParams(use_tc_tiling_on_sc=False),
    )


def _lse_body(h_ref, w_ref, b_ref, lse_ref, m_ref, s_ref):
    j = pl.program_id(0)

    @pl.when(j == 0)
    def _init():
        m_ref[...] = jnp.full_like(m_ref, _NEG_INF)
        s_ref[...] = jnp.zeros_like(s_ref)

    logits = lax.dot_general(
        h_ref[...], w_ref[...],
        (((1,), (1,)), ((), ())),
        preferred_element_type=jnp.float32,
    ) + b_ref[...]
    limit = VOCAB - j * VT
    col = lax.broadcasted_iota(jnp.int32, (1, VT), 1)
    logits = jnp.where(col < limit, logits, _NEG_INF)
    tile_max = jnp.max(logits, axis=1, keepdims=True)
    new_m = jnp.maximum(m_ref[...], tile_max)
    s_ref[...] = s_ref[...] * jnp.exp(m_ref[...] - new_m) + jnp.sum(
        jnp.exp(logits - new_m), axis=1, keepdims=True)
    m_ref[...] = new_m

    @pl.when(j == NVT - 1)
    def _fin():
        lse_ref[...] = m_ref[...] + jnp.log(s_ref[...])


def _compute_lse(h, lin_w, b2):
    return pl.pallas_call(
        _lse_body,
        grid=(NVT,),
        in_specs=[
            pl.BlockSpec((BATCH, EMBED_DIM), lambda j: (0, 0)),
            pl.BlockSpec((VT, EMBED_DIM), lambda j: (j, 0)),
            pl.BlockSpec((1, VT), lambda j: (0, j)),
        ],
        out_specs=pl.BlockSpec((BATCH, 1), lambda j: (0, 0)),
        out_shape=jax.ShapeDtypeStruct((BATCH, 1), jnp.float32),
        scratch_shapes=[
            pltpu.VMEM((BATCH, 1), jnp.float32),
            pltpu.VMEM((BATCH, 1), jnp.float32),
        ],
        compiler_params=pltpu.CompilerParams(
            dimension_semantics=("arbitrary",)),
    )(h, lin_w, b2)


def _logprob_body(h_ref, w_ref, b_ref, lse_ref, out_ref):
    logits = lax.dot_general(
        h_ref[...], w_ref[...],
        (((1,), (1,)), ((), ())),
        preferred_element_type=jnp.float32,
    ) + b_ref[...]
    out_ref[...] = logits - lse_ref[...]


def _compute_logprobs(h, lin_w, b2, lse):
    return pl.pallas_call(
        _logprob_body,
        grid=(NVT,),
        in_specs=[
            pl.BlockSpec((BATCH, EMBED_DIM), lambda j: (0, 0)),
            pl.BlockSpec((VT, EMBED_DIM), lambda j: (j, 0)),
            pl.BlockSpec((1, VT), lambda j: (0, j)),
            pl.BlockSpec((BATCH, 1), lambda j: (0, 0)),
        ],
        out_specs=pl.BlockSpec((BATCH, VT), lambda j: (0, j)),
        out_shape=jax.ShapeDtypeStruct((BATCH, VOCAB), jnp.float32),
        compiler_params=pltpu.CompilerParams(
            dimension_semantics=("arbitrary",)),
    )(h, lin_w, b2, lse)


def kernel(inputs, emb_table, lin_w, lin_b):
    idx = inputs.reshape(NW, N_CHUNKS, IDX_CHUNK)
    h = _sc_gather_sum()(idx, emb_table)
    b2 = lin_b.reshape(1, VOCAB)
    lse = _compute_lse(h, lin_w, b2)
    return _compute_logprobs(h, lin_w, b2, lse)
